# baseline (device time: 143491 ns/iter reference)
import jax
import jax.numpy as jnp
from jax import lax
from jax.experimental import pallas as pl
from jax.experimental.pallas import tpu as pltpu

N_DEV = 32
N_SUB = 4

_HAM_COORDS = (
    [(0, 0, z) for z in range(4)]
    + [(0, 1, z) for z in (3, 2, 1, 0)]
    + [(0, 2, z) for z in range(4)]
    + [(0, 3, z) for z in (3, 2, 1, 0)]
    + [(1, 3, z) for z in range(4)]
    + [(1, 2, z) for z in (3, 2, 1, 0)]
    + [(1, 1, z) for z in range(4)]
    + [(1, 0, z) for z in (3, 2, 1, 0)]
)


def _mesh_index(x, y, z):
    return z * 8 + y * 2 + (x if y % 2 == 0 else 1 - x)


_RING = [_mesh_index(*c) for c in _HAM_COORDS]
_POS = {midx: i for i, midx in enumerate(_RING)}
_NEXT = [_RING[(_POS[midx] + 1) % N_DEV] for midx in range(N_DEV)]
_PREV = [_RING[(_POS[midx] - 1) % N_DEV] for midx in range(N_DEV)]


def kernel(x, w_mat):
    m, k_per = x.shape
    _, n = w_mat.shape

    my = lax.axis_index("i")
    nxt = jnp.take(jnp.asarray(_NEXT, dtype=jnp.int32), my).reshape((1,))
    prv = jnp.take(jnp.asarray(_PREV, dtype=jnp.int32), my).reshape((1,))

    def body(nxt_ref, prv_ref, x_ref, w_ref, out_ref,
             xtbuf, wbuf, xsend, xrecv, wsend, wrecv):
        right = nxt_ref[0]
        left = prv_ref[0]

        barrier = pltpu.get_barrier_semaphore()
        for nbr in (left, right):
            pl.semaphore_signal(
                barrier, inc=1,
                device_id=(nbr,), device_id_type=pl.DeviceIdType.MESH,
            )
        pl.semaphore_wait(barrier, 2)

        xtbuf[0] = x_ref[...].astype(jnp.bfloat16).T
        wbuf[0] = w_ref[...].astype(jnp.bfloat16)

        sub = m // N_SUB
        HR = N_DEV // 2
        HL = N_DEV // 2 - 1

        def xr_desc(h, s):
            sl = pl.ds(s * sub, sub)
            return pltpu.make_async_remote_copy(
                src_ref=xtbuf.at[h, :, sl], dst_ref=xtbuf.at[h + 1, :, sl],
                send_sem=xsend.at[h, s], recv_sem=xrecv.at[h + 1, s],
                device_id=(right,), device_id_type=pl.DeviceIdType.MESH,
            )

        def wr_desc(h, s):
            sl = pl.ds(s * sub, sub)
            return pltpu.make_async_remote_copy(
                src_ref=wbuf.at[h, :, sl], dst_ref=wbuf.at[h + 1, :, sl],
                send_sem=wsend.at[h, s], recv_sem=wrecv.at[h + 1, s],
                device_id=(right,), device_id_type=pl.DeviceIdType.MESH,
            )

        def xl_desc(h, s):
            sl = pl.ds(s * sub, sub)
            return pltpu.make_async_remote_copy(
                src_ref=xtbuf.at[(N_DEV - h) % N_DEV, :, sl],
                dst_ref=xtbuf.at[N_DEV - 1 - h, :, sl],
                send_sem=xsend.at[HR + h, s],
                recv_sem=xrecv.at[N_DEV - 1 - h, s],
                device_id=(left,), device_id_type=pl.DeviceIdType.MESH,
            )

        def wl_desc(h, s):
            sl = pl.ds(s * sub, sub)
            return pltpu.make_async_remote_copy(
                src_ref=wbuf.at[(N_DEV - h) % N_DEV, :, sl],
                dst_ref=wbuf.at[N_DEV - 1 - h, :, sl],
                send_sem=wsend.at[HR + h, s],
                recv_sem=wrecv.at[N_DEV - 1 - h, s],
                device_id=(left,), device_id_type=pl.DeviceIdType.MESH,
            )

        for s in range(N_SUB):
            xr_desc(0, s).start()
            wr_desc(0, s).start()
            xl_desc(0, s).start()
            wl_desc(0, s).start()
        for h in range(HR):
            for s in range(N_SUB):
                xr_desc(h, s).wait_recv()
                if h < HR - 1:
                    xr_desc(h + 1, s).start()
                wr_desc(h, s).wait_recv()
                if h < HR - 1:
                    wr_desc(h + 1, s).start()
                if h < HL:
                    xl_desc(h, s).wait_recv()
                    if h < HL - 1:
                        xl_desc(h + 1, s).start()
                    wl_desc(h, s).wait_recv()
                    if h < HL - 1:
                        wl_desc(h + 1, s).start()
        for h in range(HR):
            for s in range(N_SUB):
                xr_desc(h, s).wait_send()
                wr_desc(h, s).wait_send()
                if h < HL:
                    xl_desc(h, s).wait_send()
                    wl_desc(h, s).wait_send()

        mb = m
        nb = n // 4
        for i in range(m // mb):
            xt = xtbuf[:, :, i * mb:(i + 1) * mb].reshape(N_DEV * k_per, mb)
            for j in range(n // nb):
                wf = wbuf[:, :, j * nb:(j + 1) * nb].reshape(N_DEV * k_per, nb)
                blk = lax.dot_general(
                    xt, wf,
                    dimension_numbers=(((0,), (0,)), ((), ())),
                    preferred_element_type=jnp.float32,
                )
                out_ref[i * mb:(i + 1) * mb, j * nb:(j + 1) * nb] = (
                    jnp.maximum(blk, 0.0)
                )

    return pl.pallas_call(
        body,
        out_shape=jax.ShapeDtypeStruct((m, n), jnp.float32),
        in_specs=[
            pl.BlockSpec(memory_space=pltpu.SMEM),
            pl.BlockSpec(memory_space=pltpu.SMEM),
            pl.BlockSpec(memory_space=pltpu.VMEM),
            pl.BlockSpec(memory_space=pltpu.VMEM),
        ],
        out_specs=pl.BlockSpec(memory_space=pltpu.VMEM),
        scratch_shapes=[
            pltpu.VMEM((N_DEV, k_per, m), jnp.bfloat16),
            pltpu.VMEM((N_DEV, k_per, n), jnp.bfloat16),
            pltpu.SemaphoreType.DMA((N_DEV, N_SUB)),
            pltpu.SemaphoreType.DMA((N_DEV, N_SUB)),
            pltpu.SemaphoreType.DMA((N_DEV, N_SUB)),
            pltpu.SemaphoreType.DMA((N_DEV, N_SUB)),
        ],
        compiler_params=pltpu.CompilerParams(
            collective_id=0,
            vmem_limit_bytes=100 * 1024 * 1024,
        ),
    )(nxt, prv, x, w_mat)


# device time: 139618 ns/iter; 1.0277x vs baseline; 1.0277x over previous
import jax
import jax.numpy as jnp
from jax import lax
from jax.experimental import pallas as pl
from jax.experimental.pallas import tpu as pltpu

N_DEV = 32
N_SUB = 2

_HAM_COORDS = (
    [(0, 0, z) for z in range(4)]
    + [(0, 1, z) for z in (3, 2, 1, 0)]
    + [(0, 2, z) for z in range(4)]
    + [(0, 3, z) for z in (3, 2, 1, 0)]
    + [(1, 3, z) for z in range(4)]
    + [(1, 2, z) for z in (3, 2, 1, 0)]
    + [(1, 1, z) for z in range(4)]
    + [(1, 0, z) for z in (3, 2, 1, 0)]
)


def _mesh_index(x, y, z):
    return z * 8 + y * 2 + (x if y % 2 == 0 else 1 - x)


_RING = [_mesh_index(*c) for c in _HAM_COORDS]
_POS = {midx: i for i, midx in enumerate(_RING)}
_NEXT = [_RING[(_POS[midx] + 1) % N_DEV] for midx in range(N_DEV)]
_PREV = [_RING[(_POS[midx] - 1) % N_DEV] for midx in range(N_DEV)]


def kernel(x, w_mat):
    m, k_per = x.shape
    _, n = w_mat.shape

    my = lax.axis_index("i")
    nxt = jnp.take(jnp.asarray(_NEXT, dtype=jnp.int32), my).reshape((1,))
    prv = jnp.take(jnp.asarray(_PREV, dtype=jnp.int32), my).reshape((1,))

    def body(nxt_ref, prv_ref, x_ref, w_ref, out_ref,
             xtbuf, wbuf, xsend, xrecv, wsend, wrecv):
        right = nxt_ref[0]
        left = prv_ref[0]

        barrier = pltpu.get_barrier_semaphore()
        for nbr in (left, right):
            pl.semaphore_signal(
                barrier, inc=1,
                device_id=(nbr,), device_id_type=pl.DeviceIdType.MESH,
            )
        pl.semaphore_wait(barrier, 2)

        xtbuf[0] = x_ref[...].astype(jnp.bfloat16).T
        wbuf[0] = w_ref[...].astype(jnp.bfloat16)

        sub = m // N_SUB
        HR = N_DEV // 2
        HL = N_DEV // 2 - 1

        def xr_desc(h, s):
            sl = pl.ds(s * sub, sub)
            return pltpu.make_async_remote_copy(
                src_ref=xtbuf.at[h, :, sl], dst_ref=xtbuf.at[h + 1, :, sl],
                send_sem=xsend.at[h, s], recv_sem=xrecv.at[h + 1, s],
                device_id=(right,), device_id_type=pl.DeviceIdType.MESH,
            )

        def wr_desc(h, s):
            sl = pl.ds(s * sub, sub)
            return pltpu.make_async_remote_copy(
                src_ref=wbuf.at[h, :, sl], dst_ref=wbuf.at[h + 1, :, sl],
                send_sem=wsend.at[h, s], recv_sem=wrecv.at[h + 1, s],
                device_id=(right,), device_id_type=pl.DeviceIdType.MESH,
            )

        def xl_desc(h, s):
            sl = pl.ds(s * sub, sub)
            return pltpu.make_async_remote_copy(
                src_ref=xtbuf.at[(N_DEV - h) % N_DEV, :, sl],
                dst_ref=xtbuf.at[N_DEV - 1 - h, :, sl],
                send_sem=xsend.at[HR + h, s],
                recv_sem=xrecv.at[N_DEV - 1 - h, s],
                device_id=(left,), device_id_type=pl.DeviceIdType.MESH,
            )

        def wl_desc(h, s):
            sl = pl.ds(s * sub, sub)
            return pltpu.make_async_remote_copy(
                src_ref=wbuf.at[(N_DEV - h) % N_DEV, :, sl],
                dst_ref=wbuf.at[N_DEV - 1 - h, :, sl],
                send_sem=wsend.at[HR + h, s],
                recv_sem=wrecv.at[N_DEV - 1 - h, s],
                device_id=(left,), device_id_type=pl.DeviceIdType.MESH,
            )

        for s in range(N_SUB):
            xr_desc(0, s).start()
            wr_desc(0, s).start()
            xl_desc(0, s).start()
            wl_desc(0, s).start()
        for h in range(HR):
            for s in range(N_SUB):
                xr_desc(h, s).wait_recv()
                if h < HR - 1:
                    xr_desc(h + 1, s).start()
                wr_desc(h, s).wait_recv()
                if h < HR - 1:
                    wr_desc(h + 1, s).start()
                if h < HL:
                    xl_desc(h, s).wait_recv()
                    if h < HL - 1:
                        xl_desc(h + 1, s).start()
                    wl_desc(h, s).wait_recv()
                    if h < HL - 1:
                        wl_desc(h + 1, s).start()
        for h in range(HR):
            for s in range(N_SUB):
                xr_desc(h, s).wait_send()
                wr_desc(h, s).wait_send()
                if h < HL:
                    xl_desc(h, s).wait_send()
                    wl_desc(h, s).wait_send()

        mb = m
        nb = n // 4
        for i in range(m // mb):
            xt = xtbuf[:, :, i * mb:(i + 1) * mb].reshape(N_DEV * k_per, mb)
            for j in range(n // nb):
                wf = wbuf[:, :, j * nb:(j + 1) * nb].reshape(N_DEV * k_per, nb)
                blk = lax.dot_general(
                    xt, wf,
                    dimension_numbers=(((0,), (0,)), ((), ())),
                    preferred_element_type=jnp.float32,
                )
                out_ref[i * mb:(i + 1) * mb, j * nb:(j + 1) * nb] = (
                    jnp.maximum(blk, 0.0)
                )

    return pl.pallas_call(
        body,
        out_shape=jax.ShapeDtypeStruct((m, n), jnp.float32),
        in_specs=[
            pl.BlockSpec(memory_space=pltpu.SMEM),
            pl.BlockSpec(memory_space=pltpu.SMEM),
            pl.BlockSpec(memory_space=pltpu.VMEM),
            pl.BlockSpec(memory_space=pltpu.VMEM),
        ],
        out_specs=pl.BlockSpec(memory_space=pltpu.VMEM),
        scratch_shapes=[
            pltpu.VMEM((N_DEV, k_per, m), jnp.bfloat16),
            pltpu.VMEM((N_DEV, k_per, n), jnp.bfloat16),
            pltpu.SemaphoreType.DMA((N_DEV, N_SUB)),
            pltpu.SemaphoreType.DMA((N_DEV, N_SUB)),
            pltpu.SemaphoreType.DMA((N_DEV, N_SUB)),
            pltpu.SemaphoreType.DMA((N_DEV, N_SUB)),
        ],
        compiler_params=pltpu.CompilerParams(
            collective_id=0,
            vmem_limit_bytes=100 * 1024 * 1024,
        ),
    )(nxt, prv, x, w_mat)
